# baseline (speedup 1.0000x reference)
import jax
import jax.numpy as jnp
from jax.experimental import pallas as pl
from jax.experimental.pallas import tpu as pltpu

_LOG2E = 1.4426950408889634


def _layer(h, beta_log2e, ones_col):
    D = h.shape[1]
    nrm2 = jnp.sum(h * h, axis=1, keepdims=True)
    r = jax.lax.rsqrt(jnp.maximum(nrm2, 1e-24))
    hn = (h * r).astype(jnp.bfloat16)
    hs = (h * (r * beta_log2e)).astype(jnp.bfloat16)
    s = jax.lax.dot_general(
        hs, hn, (((1,), (1,)), ((), ())),
        preferred_element_type=jnp.float32)
    p = jnp.exp2(s.astype(jnp.bfloat16))
    ha = jnp.concatenate([h.astype(jnp.bfloat16), ones_col], axis=1)
    u = jax.lax.dot_general(
        p, ha, (((1,), (0,)), ((), ())),
        preferred_element_type=jnp.float32)
    return u[:, :D] / u[:, D:D + 1]


def _body(betas_ref, x_ref, feat_ref, out_ref, xb_ref, hs1_ref, hn1_ref,
          ha1_ref):
    j = pl.program_id(0)

    @pl.when(j == 0)
    def _prep():
        xb_ref[:] = x_ref[:].astype(jnp.bfloat16)
        h = feat_ref[:]
        C, D = h.shape
        pad = hs1_ref.shape[0] - C
        ones_col = jnp.ones((C, 1), dtype=jnp.bfloat16)
        h1 = _layer(h, betas_ref[0] * _LOG2E, ones_col)
        nrm2 = jnp.sum(h1 * h1, axis=1, keepdims=True)
        r = jax.lax.rsqrt(jnp.maximum(nrm2, 1e-24))
        hn1_ref[:] = (h1 * r).astype(jnp.bfloat16)
        ha1_ref[:] = jnp.concatenate(
            [h1.astype(jnp.bfloat16), ones_col], axis=1)
        hs1_ref[pl.ds(0, C), :] = (
            h1 * (r * (betas_ref[1] * _LOG2E))).astype(jnp.bfloat16)
        hs1_ref[pl.ds(C, pad), :] = jnp.zeros((pad, D), jnp.bfloat16)

    D = xb_ref.shape[1]
    hs_j = hs1_ref[pl.ds(j * 128, 128), :]
    s_j = jax.lax.dot_general(
        hs_j, hn1_ref[:], (((1,), (1,)), ((), ())),
        preferred_element_type=jnp.float32)
    p_j = jnp.exp2(s_j.astype(jnp.bfloat16))
    u_j = jax.lax.dot_general(
        p_j, ha1_ref[:], (((1,), (0,)), ((), ())),
        preferred_element_type=jnp.float32)
    y_j = (u_j[:, :D] / u_j[:, D:D + 1]).astype(jnp.bfloat16)
    out_ref[:] = jax.lax.dot_general(
        xb_ref[:], y_j, (((1,), (1,)), ((), ())),
        preferred_element_type=jnp.float32)


def kernel(x, feat, src, dst, beta0, beta1):
    del src, dst
    B, D = x.shape
    C = feat.shape[0]
    nj = (C + 127) // 128
    cpad = nj * 128
    betas = jnp.stack([jnp.asarray(beta0, jnp.float32),
                       jnp.asarray(beta1, jnp.float32)])
    grid_spec = pltpu.PrefetchScalarGridSpec(
        num_scalar_prefetch=1,
        grid=(nj,),
        in_specs=[
            pl.BlockSpec((B, D), lambda j, betas: (0, 0)),
            pl.BlockSpec((C, D), lambda j, betas: (0, 0)),
        ],
        out_specs=pl.BlockSpec((B, 128), lambda j, betas: (0, j)),
        scratch_shapes=[
            pltpu.VMEM((B, D), jnp.bfloat16),
            pltpu.VMEM((cpad, D), jnp.bfloat16),
            pltpu.VMEM((C, D), jnp.bfloat16),
            pltpu.VMEM((C, D + 1), jnp.bfloat16),
        ],
    )
    return pl.pallas_call(
        _body,
        grid_spec=grid_spec,
        out_shape=jax.ShapeDtypeStruct((B, C), jnp.float32),
    )(betas, x, feat)


# speedup vs baseline: 1.0531x; 1.0531x over previous
import jax
import jax.numpy as jnp
from jax.experimental import pallas as pl
from jax.experimental.pallas import tpu as pltpu


def _body(betas_ref, x_ref, feat_ref, out_ref, y_ref):
    @pl.when(pl.program_id(0) == 0)
    def _compute_y():
        h = feat_ref[:]
        C, D = h.shape
        ones_col = jnp.ones((C, 1), dtype=jnp.bfloat16)
        for i in range(2):
            beta = betas_ref[i]
            nrm2 = jnp.sum(h * h, axis=1, keepdims=True)
            r = jax.lax.rsqrt(jnp.maximum(nrm2, 1e-24))
            hn = (h * r).astype(jnp.bfloat16)
            hs = (h * (r * (beta * 1.4426950408889634))).astype(jnp.bfloat16)
            s = jax.lax.dot_general(
                hs, hn, (((1,), (1,)), ((), ())),
                preferred_element_type=jnp.float32)
            p = jnp.exp2(s.astype(jnp.bfloat16))
            ha = jnp.concatenate([h.astype(jnp.bfloat16), ones_col], axis=1)
            u = jax.lax.dot_general(
                p, ha, (((1,), (0,)), ((), ())),
                preferred_element_type=jnp.float32)
            h = u[:, :D] / u[:, D:D + 1]
        y_ref[:] = h.astype(jnp.bfloat16)

    out_ref[:] = jax.lax.dot_general(
        x_ref[:].astype(jnp.bfloat16), y_ref[:], (((1,), (1,)), ((), ())),
        preferred_element_type=jnp.float32)


def kernel(x, feat, src, dst, beta0, beta1):
    del src, dst
    B, D = x.shape
    C = feat.shape[0]
    BB = 1024
    nb = B // BB
    betas = jnp.stack([jnp.asarray(beta0, jnp.float32),
                       jnp.asarray(beta1, jnp.float32)])
    grid_spec = pltpu.PrefetchScalarGridSpec(
        num_scalar_prefetch=1,
        grid=(nb,),
        in_specs=[
            pl.BlockSpec((BB, D), lambda i, betas: (i, 0)),
            pl.BlockSpec((C, D), lambda i, betas: (0, 0)),
        ],
        out_specs=pl.BlockSpec((BB, C), lambda i, betas: (i, 0)),
        scratch_shapes=[pltpu.VMEM((C, D), jnp.bfloat16)],
    )
    return pl.pallas_call(
        _body,
        grid_spec=grid_spec,
        out_shape=jax.ShapeDtypeStruct((B, C), jnp.float32),
    )(betas, x, feat)
